# baseline (speedup 1.0000x reference)
import functools

import jax
import jax.numpy as jnp
from jax import lax
from jax.experimental import pallas as pl
from jax.experimental.pallas import tpu as pltpu
from jax.experimental.pallas import tpu_sc as plsc

_EMB = 16
_MAX_TILE_ELEMS = 65536


@functools.lru_cache(maxsize=None)
def _build_fill(n_rows: int):
    info = plsc.get_sparse_core_info()
    nw = info.num_cores * info.num_subcores
    total_e = n_rows * _EMB

    q_rows = n_rows // nw
    left_rows = n_rows - q_rows * nw
    q_e = q_rows * _EMB

    tile_rows = min(q_rows if q_rows > 0 else 1, _MAX_TILE_ELEMS // _EMB)
    tile_e = tile_rows * _EMB
    n_full = q_rows // tile_rows if tile_rows else 0
    rem_e = (q_rows - n_full * tile_rows) * _EMB
    left_e = left_rows * _EMB

    mesh = plsc.VectorSubcoreMesh(core_axis_name="c", subcore_axis_name="s")

    @functools.partial(
        pl.kernel,
        mesh=mesh,
        out_type=jax.ShapeDtypeStruct((total_e,), jnp.float32),
        scratch_types=[
            pltpu.VMEM((tile_e,), jnp.float32),
            pltpu.SemaphoreType.DMA,
        ],
    )
    def fill(w_hbm, out_hbm, buf, sem):
        wid = lax.axis_index("s") * info.num_cores + lax.axis_index("c")
        base_e = wid * q_e

        pltpu.sync_copy(w_hbm, buf.at[pl.ds(0, _EMB)])
        w = buf[pl.ds(0, _EMB)]
        unroll = 8
        n_steps = (tile_rows - 1) // unroll

        def body(i, carry):
            b = _EMB + i * (_EMB * unroll)
            for k in range(unroll):
                buf[pl.ds(b + k * _EMB, _EMB)] = w
            return carry

        lax.fori_loop(0, n_steps, body, 0)
        for r in range(1 + n_steps * unroll, tile_rows):
            buf[pl.ds(r * _EMB, _EMB)] = w

        copies = []
        for j in range(n_full):
            c = pltpu.make_async_copy(
                buf, out_hbm.at[pl.ds(base_e + j * tile_e, tile_e)], sem)
            c.start()
            copies.append(c)
        if rem_e:
            c = pltpu.make_async_copy(
                buf.at[pl.ds(0, rem_e)],
                out_hbm.at[pl.ds(base_e + n_full * tile_e, rem_e)], sem)
            c.start()
            copies.append(c)
        if left_e:
            @pl.when(wid == nw - 1)
            def _():
                pltpu.make_async_copy(
                    buf.at[pl.ds(0, left_e)],
                    out_hbm.at[pl.ds(nw * q_e, left_e)], sem).start()
        for c in copies:
            c.wait()
        if left_e:
            @pl.when(wid == nw - 1)
            def _():
                pltpu.make_async_copy(
                    buf.at[pl.ds(0, left_e)],
                    out_hbm.at[pl.ds(nw * q_e, left_e)], sem).wait()

    return fill


def kernel(edge_index, weight):
    n = edge_index.shape[1]
    out_flat = _build_fill(n)(weight.reshape(_EMB).astype(jnp.float32))
    return out_flat.reshape(n, _EMB)
